# baseline (speedup 1.0000x reference)
import jax
import jax.numpy as jnp
from jax import lax
from jax.experimental import pallas as pl
from jax.experimental.pallas import tpu as pltpu
from jax.experimental.pallas import tpu_sc as plsc

N_COARSE = 128
N_FINE = 64
B = 100000
NW = 32
RBLK = 80
NBLK_TOTAL = B // RBLK
NPAIR = (NBLK_TOTAL // NW + 2) // 2
INV = 1.0 / N_COARSE


def _sc_body(rays_hbm, w_hbm, uc_hbm, uf_hbm, uf2_hbm, out_hbm,
             raysA, wA, ucA, ufA, uf2A,
             raysB, wB, ucB, ufB, uf2B,
             cdf_v, out_v, semA, semB):
    c = lax.axis_index("c")
    s = lax.axis_index("s")
    wid = s * 2 + c
    nb = (NBLK_TOTAL - wid + NW - 1) // NW

    bufsA = (raysA, wA, ucA, ufA, uf2A)
    bufsB = (raysB, wB, ucB, ufB, uf2B)

    def copy_pairs(k, bufs):
        base = (wid + k * NW) * RBLK
        rays_v, w_v, uc_v, uf_v, uf2_v = bufs
        return (
            (rays_hbm.at[pl.ds(base * 8, RBLK * 8)], rays_v.at[pl.ds(0, RBLK * 8)]),
            (w_hbm.at[pl.ds(base, RBLK)], w_v),
            (uc_hbm.at[pl.ds(base, RBLK)], uc_v),
            (uf_hbm.at[pl.ds(base, RBLK)], uf_v),
            (uf2_hbm.at[pl.ds(base, RBLK)], uf2_v),
        )

    def issue(k, bufs, sem):
        for src, dst in copy_pairs(k, bufs):
            pltpu.async_copy(src, dst, sem)

    def drain(k, bufs, sem):
        for src, dst in copy_pairs(k, bufs):
            pltpu.make_async_copy(src, dst, sem).wait()

    def compute(k, bufs):
        rays_v, w_v, uc_v, uf_v, uf2_v = bufs
        base = (wid + k * NW) * RBLK

        @plsc.parallel_loop(0, RBLK, 1, unroll=2)
        def ray_body(r):
            rvec = rays_v[pl.ds(r * 8, 16)]
            near = rvec[6]
            far = rvec[7]
            d128 = (far - near) * jnp.float32(INV)
            scans = []
            for k8 in range(8):
                wk = w_v[r, pl.ds(16 * k8, 16)] + jnp.float32(1e-5)
                scans.append(plsc.cumsum(wk))
            total = jnp.float32(0.0)
            c63 = jnp.float32(0.0)
            for k8 in range(8):
                ck = scans[k8] + total
                cdf_v[pl.ds(r * N_COARSE + 16 * k8, 16)] = ck
                total = total + scans[k8][15]
                if k8 == 3:
                    c63 = total
            jf16 = lax.iota(jnp.int32, 16).astype(jnp.float32)
            for k8 in range(8):
                basek = (jf16 + jnp.float32(16 * k8)) * d128 + near
                out_v[r, pl.ds(16 * k8, 16)] = uc_v[r, pl.ds(16 * k8, 16)] * d128 + basek

            cbase = jnp.full((16,), r * N_COARSE, jnp.int32)
            for q in range(4):
                v = uf_v[r, pl.ds(16 * q, 16)] * total
                lo = jnp.zeros((16,), jnp.int32)
                hi = jnp.full((16,), N_COARSE, jnp.int32)
                for _ in range(7):
                    mid = (lo + hi) >> 1
                    cg = plsc.load_gather(cdf_v, [cbase + mid])
                    pred = cg <= v
                    lo = jnp.where(pred, mid + 1, lo)
                    hi = jnp.where(pred, hi, mid)
                zs = lo.astype(jnp.float32) + uf2_v[r, pl.ds(16 * q, 16)]
                out_v[r, pl.ds(N_COARSE + 16 * q, 16)] = zs * d128 + near

        pltpu.sync_copy(out_v, out_hbm.at[pl.ds(base, RBLK)])

    issue(0, bufsA, semA)

    def pair(j, carry):
        b0, b1, b2 = 2 * j, 2 * j + 1, 2 * j + 2

        @pl.when(b1 < nb)
        def _():
            issue(b1, bufsB, semB)

        @pl.when(b0 < nb)
        def _():
            drain(b0, bufsA, semA)
            compute(b0, bufsA)

        @pl.when(b2 < nb)
        def _():
            issue(b2, bufsA, semA)

        @pl.when(b1 < nb)
        def _():
            drain(b1, bufsB, semB)
            compute(b1, bufsB)

        return carry

    lax.fori_loop(0, NPAIR, pair, 0)


def kernel(rays, weights, u_coarse, u_fine, u_fine2):
    rays_flat = rays.reshape(-1)
    mesh = plsc.VectorSubcoreMesh(core_axis_name="c", subcore_axis_name="s")
    in_bufs = [
        pltpu.VMEM((RBLK * 8 + 16,), jnp.float32),
        pltpu.VMEM((RBLK, N_COARSE), jnp.float32),
        pltpu.VMEM((RBLK, N_COARSE), jnp.float32),
        pltpu.VMEM((RBLK, N_FINE), jnp.float32),
        pltpu.VMEM((RBLK, N_FINE), jnp.float32),
    ]
    f = pl.kernel(
        _sc_body,
        mesh=mesh,
        compiler_params=pltpu.CompilerParams(
            use_tc_tiling_on_sc=True, needs_layout_passes=False
        ),
        out_type=jax.ShapeDtypeStruct((B, N_COARSE + N_FINE), jnp.float32),
        scratch_types=in_bufs + in_bufs + [
            pltpu.VMEM((RBLK * N_COARSE,), jnp.float32),
            pltpu.VMEM((RBLK, N_COARSE + N_FINE), jnp.float32),
            pltpu.SemaphoreType.DMA,
            pltpu.SemaphoreType.DMA,
        ],
    )
    return f(rays_flat, weights, u_coarse, u_fine, u_fine2)


# speedup vs baseline: 1.1071x; 1.1071x over previous
import jax
import jax.numpy as jnp
from jax import lax
from jax.experimental import pallas as pl
from jax.experimental.pallas import tpu as pltpu
from jax.experimental.pallas import tpu_sc as plsc

N_COARSE = 128
N_FINE = 64
B = 100000
NW = 32
RBLK = 40
NBLK_TOTAL = B // RBLK
NPAIR = (NBLK_TOTAL // NW + 2) // 2
INV = 1.0 / N_COARSE


def _sc_body(near_hbm, far_hbm, w_hbm, uc_hbm, uf_hbm, uf2_hbm, out_hbm,
             nearA, farA, wA, ucA, ufA, uf2A,
             nearB, farB, wB, ucB, ufB, uf2B,
             cdf_v, outA, outB, semA, semB, semOA, semOB):
    c = lax.axis_index("c")
    s = lax.axis_index("s")
    wid = s * 2 + c
    nb = (NBLK_TOTAL - wid + NW - 1) // NW

    bufsA = (nearA, farA, wA, ucA, ufA, uf2A)
    bufsB = (nearB, farB, wB, ucB, ufB, uf2B)

    def copy_pairs(k, bufs):
        base = (wid + k * NW) * RBLK
        near_v, far_v, w_v, uc_v, uf_v, uf2_v = bufs
        return (
            (near_hbm.at[pl.ds(base, RBLK)], near_v.at[pl.ds(0, RBLK)]),
            (far_hbm.at[pl.ds(base, RBLK)], far_v.at[pl.ds(0, RBLK)]),
            (w_hbm.at[pl.ds(base, RBLK)], w_v),
            (uc_hbm.at[pl.ds(base, RBLK)], uc_v),
            (uf_hbm.at[pl.ds(base, RBLK)], uf_v),
            (uf2_hbm.at[pl.ds(base, RBLK)], uf2_v),
        )

    def issue(k, bufs, sem):
        for src, dst in copy_pairs(k, bufs):
            pltpu.async_copy(src, dst, sem)

    def drain(k, bufs, sem):
        for src, dst in copy_pairs(k, bufs):
            pltpu.make_async_copy(src, dst, sem).wait()

    def out_copy(k, out_v):
        base = (wid + k * NW) * RBLK
        return (out_v, out_hbm.at[pl.ds(base, RBLK)])

    def compute(k, bufs, out_v, semO):
        near_v, far_v, w_v, uc_v, uf_v, uf2_v = bufs
        base = (wid + k * NW) * RBLK

        @pl.when(k >= 2)
        def _():
            src, dst = out_copy(k - 2, out_v)
            pltpu.make_async_copy(src, dst, semO).wait()

        @plsc.parallel_loop(0, RBLK, 1, unroll=2)
        def ray_body(r):
            near = near_v[pl.ds(r, 16)][0]
            far = far_v[pl.ds(r, 16)][0]
            d128 = (far - near) * jnp.float32(INV)
            scans = []
            for k8 in range(8):
                wk = w_v[r, pl.ds(16 * k8, 16)] + jnp.float32(1e-5)
                scans.append(plsc.cumsum(wk))
            total = jnp.float32(0.0)
            c63 = jnp.float32(0.0)
            for k8 in range(8):
                ck = scans[k8] + total
                cdf_v[pl.ds(r * N_COARSE + 16 * k8, 16)] = ck
                total = total + scans[k8][15]
                if k8 == 3:
                    c63 = total
            jf16 = lax.iota(jnp.int32, 16).astype(jnp.float32)
            for k8 in range(8):
                basek = (jf16 + jnp.float32(16 * k8)) * d128 + near
                out_v[r, pl.ds(16 * k8, 16)] = uc_v[r, pl.ds(16 * k8, 16)] * d128 + basek

            cbase = jnp.full((16,), r * N_COARSE, jnp.int32)
            for q in range(4):
                v = uf_v[r, pl.ds(16 * q, 16)] * total
                lo = jnp.zeros((16,), jnp.int32)
                hi = jnp.full((16,), N_COARSE, jnp.int32)
                for _ in range(7):
                    mid = (lo + hi) >> 1
                    cg = plsc.load_gather(cdf_v, [cbase + mid])
                    pred = cg <= v
                    lo = jnp.where(pred, mid + 1, lo)
                    hi = jnp.where(pred, hi, mid)
                zs = lo.astype(jnp.float32) + uf2_v[r, pl.ds(16 * q, 16)]
                out_v[r, pl.ds(N_COARSE + 16 * q, 16)] = zs * d128 + near

        src, dst = out_copy(k, out_v)
        pltpu.async_copy(src, dst, semO)

    issue(0, bufsA, semA)

    def pair(j, carry):
        b0, b1, b2 = 2 * j, 2 * j + 1, 2 * j + 2

        @pl.when(b1 < nb)
        def _():
            issue(b1, bufsB, semB)

        @pl.when(b0 < nb)
        def _():
            drain(b0, bufsA, semA)
            compute(b0, bufsA, outA, semOA)

        @pl.when(b2 < nb)
        def _():
            issue(b2, bufsA, semA)

        @pl.when(b1 < nb)
        def _():
            drain(b1, bufsB, semB)
            compute(b1, bufsB, outB, semOB)

        return carry

    lax.fori_loop(0, NPAIR, pair, 0)

    @pl.when(nb % 2 == 1)
    def _():
        src, dst = out_copy(nb - 1, outA)
        pltpu.make_async_copy(src, dst, semOA).wait()
        src, dst = out_copy(nb - 2, outB)
        pltpu.make_async_copy(src, dst, semOB).wait()

    @pl.when(nb % 2 == 0)
    def _():
        src, dst = out_copy(nb - 1, outB)
        pltpu.make_async_copy(src, dst, semOB).wait()
        src, dst = out_copy(nb - 2, outA)
        pltpu.make_async_copy(src, dst, semOA).wait()


def kernel(rays, weights, u_coarse, u_fine, u_fine2):
    near = rays[:, 6]
    far = rays[:, 7]
    mesh = plsc.VectorSubcoreMesh(core_axis_name="c", subcore_axis_name="s")
    in_bufs = [
        pltpu.VMEM((RBLK + 16,), jnp.float32),
        pltpu.VMEM((RBLK + 16,), jnp.float32),
        pltpu.VMEM((RBLK, N_COARSE), jnp.float32),
        pltpu.VMEM((RBLK, N_COARSE), jnp.float32),
        pltpu.VMEM((RBLK, N_FINE), jnp.float32),
        pltpu.VMEM((RBLK, N_FINE), jnp.float32),
    ]
    f = pl.kernel(
        _sc_body,
        mesh=mesh,
        compiler_params=pltpu.CompilerParams(
            use_tc_tiling_on_sc=True, needs_layout_passes=False
        ),
        out_type=jax.ShapeDtypeStruct((B, N_COARSE + N_FINE), jnp.float32),
        scratch_types=in_bufs + in_bufs + [
            pltpu.VMEM((RBLK * N_COARSE,), jnp.float32),
            pltpu.VMEM((RBLK, N_COARSE + N_FINE), jnp.float32),
            pltpu.VMEM((RBLK, N_COARSE + N_FINE), jnp.float32),
            pltpu.SemaphoreType.DMA,
            pltpu.SemaphoreType.DMA,
            pltpu.SemaphoreType.DMA,
            pltpu.SemaphoreType.DMA,
        ],
    )
    return f(near, far, weights, u_coarse, u_fine, u_fine2)
